# baseline (speedup 1.0000x reference)
import functools

import jax
import jax.numpy as jnp
from jax.experimental import pallas as pl
from jax.experimental.pallas import tpu as pltpu

KNN = 20
EPS = 1e-5


def _knn_idx(x, k):
    xx = jnp.sum(x * x, axis=1)
    inner = jnp.einsum('bdn,bdm->bnm', x, x)
    neg_dist = 2.0 * inner - xx[:, :, None] - xx[:, None, :]
    _, idx = jax.lax.top_k(neg_dist, k)
    return idx


def _graph_feature(x, idx):
    xt = jnp.transpose(x, (0, 2, 1))
    nbrs = jax.vmap(lambda xb, ib: xb[ib])(xt, idx)
    center = xt[:, :, None, :]
    return jnp.concatenate([nbrs - center, jnp.broadcast_to(center, nbrs.shape)], axis=-1)


def _conv_bn_lrelu(y, W, g, b):
    y = jnp.einsum('...c,oc->...o', y, W)
    axes = tuple(range(y.ndim - 1))
    mu = jnp.mean(y, axis=axes, keepdims=True)
    var = jnp.var(y, axis=axes, keepdims=True)
    y = (y - mu) / jnp.sqrt(var + EPS)
    y = y * g + b
    return jnp.where(y > 0, y, 0.2 * y)


def _edge_conv(x, W, g, b):
    idx = _knn_idx(x, KNN)
    feat = _graph_feature(x, idx)
    y = _conv_bn_lrelu(feat, W, g, b)
    y = jnp.max(y, axis=2)
    return jnp.transpose(y, (0, 2, 1))


def _multi_edge_conv(x, layers):
    idx = _knn_idx(x, KNN)
    feat = _graph_feature(x, idx)
    for (W, g, b) in layers:
        feat = _conv_bn_lrelu(feat, W, g, b)
    y = jnp.max(feat, axis=2)
    return jnp.transpose(y, (0, 2, 1))


def _encoder(x, p):
    x1 = _multi_edge_conv(x, [(p['enc0_W0'], p['enc0_g0'], p['enc0_b0']),
                              (p['enc0_W1'], p['enc0_g1'], p['enc0_b1'])])
    x2 = _multi_edge_conv(x1, [(p['enc1_W0'], p['enc1_g0'], p['enc1_b0']),
                               (p['enc1_W1'], p['enc1_g1'], p['enc1_b1'])])
    x3 = _edge_conv(x2, p['enc2_W'], p['enc2_g'], p['enc2_b'])
    x4 = _edge_conv(x3, p['enc3_W'], p['enc3_g'], p['enc3_b'])
    return jnp.concatenate([x1, x2, x3, x4], axis=1)


def _tail(x, p):
    y = jnp.einsum('bcn,oc->bon', x, p['tail_W'])
    mu = jnp.mean(y, axis=(0, 2), keepdims=True)
    var = jnp.var(y, axis=(0, 2), keepdims=True)
    y = (y - mu) / jnp.sqrt(var + EPS)
    y = y * p['tail_g'][None, :, None] + p['tail_b'][None, :, None]
    return jnp.where(y > 0, y, 0.2 * y)


def _final_proj_body(z_ref, w_ref, bias_ref, out_ref):
    z = z_ref[0]
    w = w_ref[...]
    out_ref[0] = jnp.dot(w, z, preferred_element_type=jnp.float32) \
        + bias_ref[...][:, None]


def _final_proj(z, W, bias):
    B, C, N = z.shape
    O = W.shape[0]
    return pl.pallas_call(
        _final_proj_body,
        grid=(B,),
        in_specs=[
            pl.BlockSpec((1, C, N), lambda b: (b, 0, 0)),
            pl.BlockSpec((O, C), lambda b: (0, 0)),
            pl.BlockSpec((O,), lambda b: (0,)),
        ],
        out_specs=pl.BlockSpec((1, O, N), lambda b: (b, 0, 0)),
        out_shape=jax.ShapeDtypeStruct((B, O, N), jnp.float32),
    )(z, W, bias)


def kernel(x, y, params):
    p = params
    x1 = _tail(_encoder(x, p), p)
    x2 = _tail(_encoder(y, p), p)
    z = jnp.concatenate([x1, x2], axis=1)
    z = _edge_conv(z, p['dec0_W'], p['dec0_g'], p['dec0_b'])
    z = _edge_conv(z, p['dec1_W'], p['dec1_g'], p['dec1_b'])
    return _final_proj(z, p['dec2_W'], p['dec2_bias'])
